# baseline (speedup 1.0000x reference)
import functools

import jax
import jax.numpy as jnp
from jax import lax
from jax.experimental import pallas as pl
from jax.experimental.pallas import tpu as pltpu
from jax.experimental.pallas import tpu_sc as plsc

CHUNK = 128
NBUF = 3


def _make_sc_gather(n_chunks_per_worker, hidden, nc, ns):
    nw = nc * ns
    rows_per_worker = n_chunks_per_worker * CHUNK
    mesh = plsc.VectorSubcoreMesh(core_axis_name="c", subcore_axis_name="s")

    @functools.partial(
        pl.kernel,
        mesh=mesh,
        out_type=jax.ShapeDtypeStruct(
            (nw * rows_per_worker, hidden), jnp.float32
        ),
        scratch_types=[
            pltpu.VMEM((n_chunks_per_worker, CHUNK), jnp.int32),
            pltpu.VMEM((NBUF, CHUNK, hidden), jnp.float32),
            pltpu.SemaphoreType.DMA,
        ],
    )
    def sc_gather(idx_hbm, table_hbm, out_hbm, idx_v, rows_v, sem):
        wid = lax.axis_index("s") * nc + lax.axis_index("c")
        base = wid * rows_per_worker
        pltpu.sync_copy(idx_hbm.at[wid], idx_v)

        def fire(j):
            return pltpu.async_copy(
                table_hbm.at[idx_v.at[j]], rows_v.at[j % NBUF], sem
            )

        handles = {}
        for j in range(min(NBUF - 1, n_chunks_per_worker)):
            handles[j] = fire(j)
        for j in range(n_chunks_per_worker):
            handles[j].wait()
            nxt = j + NBUF - 1
            if nxt < n_chunks_per_worker:
                handles[nxt] = fire(nxt)
            pltpu.sync_copy(
                rows_v.at[j % NBUF],
                out_hbm.at[pl.ds(base + j * CHUNK, CHUNK)],
            )

    return sc_gather


def _relu_ff_body(x_ref, w2_ref, b2_ref, o_ref):
    o_ref[...] = jnp.maximum(
        jnp.dot(x_ref[...], w2_ref[...], preferred_element_type=jnp.float32)
        + b2_ref[...],
        0.0,
    )


def _out_softmax_body(h_ref, wout_ref, bout_ref, o_ref):
    logits = (
        jnp.dot(h_ref[...], wout_ref[...], preferred_element_type=jnp.float32)
        + bout_ref[...]
    )
    m = jnp.max(logits, axis=-1, keepdims=True)
    e = jnp.exp(logits - m)
    o_ref[...] = e * (1.0 / jnp.sum(e, axis=-1, keepdims=True))


def kernel(input_seq, output_seq, emb, W1, b1, W2, b2, Wout, bout):
    del input_seq, W1, b1

    batch, out_len = output_seq.shape
    n_rows = batch * out_len
    vocab, embed_dim = emb.shape
    hidden = W2.shape[1]
    out_vocab = Wout.shape[1]

    info = plsc.get_sparse_core_info()
    nc, ns = info.num_cores, info.num_subcores
    nw = nc * ns

    pre_rows = 1000
    htable = pl.pallas_call(
        _relu_ff_body,
        grid=(vocab // pre_rows,),
        in_specs=[
            pl.BlockSpec((pre_rows, embed_dim), lambda i: (i, 0)),
            pl.BlockSpec((embed_dim, hidden), lambda i: (0, 0)),
            pl.BlockSpec((1, hidden), lambda i: (0, 0)),
        ],
        out_specs=pl.BlockSpec((pre_rows, hidden), lambda i: (i, 0)),
        out_shape=jax.ShapeDtypeStruct((vocab, hidden), jnp.float32),
    )(emb, W2, b2.reshape(1, hidden))

    idx = output_seq.reshape(-1).astype(jnp.int32)
    per_worker = -(-n_rows // (nw * CHUNK)) * CHUNK
    n_pad = nw * per_worker
    idx = jnp.pad(idx, (0, n_pad - n_rows))
    idx = idx.reshape(nw, per_worker // CHUNK, CHUNK)

    gathered = _make_sc_gather(per_worker // CHUNK, hidden, nc, ns)(
        idx, htable
    )

    tile_rows = 512
    out = pl.pallas_call(
        _out_softmax_body,
        grid=(n_rows // tile_rows,),
        in_specs=[
            pl.BlockSpec((tile_rows, hidden), lambda i: (i, 0)),
            pl.BlockSpec((hidden, out_vocab), lambda i: (0, 0)),
            pl.BlockSpec((1, out_vocab), lambda i: (0, 0)),
        ],
        out_specs=pl.BlockSpec((tile_rows, out_vocab), lambda i: (i, 0)),
        out_shape=jax.ShapeDtypeStruct((n_rows, out_vocab), jnp.float32),
    )(gathered, Wout, bout.reshape(1, out_vocab))

    return out.reshape(batch, out_len, out_vocab)


# speedup vs baseline: 2.1000x; 2.1000x over previous
import functools

import jax
import jax.numpy as jnp
from jax import lax
from jax.experimental import pallas as pl
from jax.experimental.pallas import tpu as pltpu
from jax.experimental.pallas import tpu_sc as plsc

CHUNK = 128
NBUF = 3


def _make_sc_gather(n_chunks_per_worker, hidden, nc, ns):
    nw = nc * ns
    rows_per_worker = n_chunks_per_worker * CHUNK
    mesh = plsc.VectorSubcoreMesh(core_axis_name="c", subcore_axis_name="s")

    @functools.partial(
        pl.kernel,
        mesh=mesh,
        out_type=jax.ShapeDtypeStruct(
            (nw * rows_per_worker, hidden), jnp.float32
        ),
        scratch_types=[
            pltpu.VMEM((n_chunks_per_worker, CHUNK), jnp.int32),
            pltpu.VMEM((NBUF, CHUNK, hidden), jnp.float32),
            pltpu.SemaphoreType.DMA,
        ],
    )
    def sc_gather(idx_hbm, table_hbm, out_hbm, idx_v, rows_v, sem):
        wid = lax.axis_index("s") * nc + lax.axis_index("c")
        base = wid * rows_per_worker
        pltpu.sync_copy(idx_hbm.at[wid], idx_v)

        def fire(j):
            return pltpu.async_copy(
                table_hbm.at[idx_v.at[j]], rows_v.at[j % NBUF], sem
            )

        handles = {}
        for j in range(min(NBUF - 1, n_chunks_per_worker)):
            handles[j] = fire(j)
        for j in range(n_chunks_per_worker):
            handles[j].wait()
            nxt = j + NBUF - 1
            if nxt < n_chunks_per_worker:
                handles[nxt] = fire(nxt)
            pltpu.sync_copy(
                rows_v.at[j % NBUF],
                out_hbm.at[pl.ds(base + j * CHUNK, CHUNK)],
            )

    return sc_gather


def _relu_ff_t_body(xt_ref, w2_ref, b2_ref, o_ref):
    h = lax.dot_general(
        xt_ref[...],
        w2_ref[...],
        (((0,), (0,)), ((), ())),
        preferred_element_type=jnp.float32,
    )
    o_ref[...] = jnp.maximum(h + b2_ref[...], 0.0)


def _out_softmax_t_body(h_ref, woutt_ref, boutt_ref, o_ref):
    logits = lax.dot_general(
        woutt_ref[...],
        h_ref[...],
        (((1,), (1,)), ((), ())),
        preferred_element_type=jnp.float32,
    )
    logits = logits + boutt_ref[...]
    m = jnp.max(logits, axis=0, keepdims=True)
    e = jnp.exp(logits - m)
    o_ref[...] = (e * (1.0 / jnp.sum(e, axis=0, keepdims=True)))[None]


def kernel(input_seq, output_seq, emb, W1, b1, W2, b2, Wout, bout):
    del input_seq, W1, b1

    batch, out_len = output_seq.shape
    n_rows = batch * out_len
    vocab, embed_dim = emb.shape
    hidden = W2.shape[1]
    out_vocab = Wout.shape[1]

    info = plsc.get_sparse_core_info()
    nc, ns = info.num_cores, info.num_subcores
    nw = nc * ns

    embT = jnp.transpose(emb)
    pre_rows = 2048
    htable = pl.pallas_call(
        _relu_ff_t_body,
        grid=(-(-vocab // pre_rows),),
        in_specs=[
            pl.BlockSpec((embed_dim, pre_rows), lambda i: (0, i)),
            pl.BlockSpec((embed_dim, hidden), lambda i: (0, 0)),
            pl.BlockSpec((1, hidden), lambda i: (0, 0)),
        ],
        out_specs=pl.BlockSpec((pre_rows, hidden), lambda i: (i, 0)),
        out_shape=jax.ShapeDtypeStruct((vocab, hidden), jnp.float32),
    )(embT, W2, b2.reshape(1, hidden))

    idx = jnp.transpose(output_seq).reshape(-1).astype(jnp.int32)
    per_worker = -(-n_rows // (nw * CHUNK)) * CHUNK
    n_pad = nw * per_worker
    idx = jnp.pad(idx, (0, n_pad - n_rows))
    idx = idx.reshape(nw, per_worker // CHUNK, CHUNK)

    gathered = _make_sc_gather(per_worker // CHUNK, hidden, nc, ns)(
        idx, htable
    )

    tile_rows = 512
    b_tiles = batch // tile_rows
    outT = pl.pallas_call(
        _out_softmax_t_body,
        grid=(out_len, b_tiles),
        in_specs=[
            pl.BlockSpec(
                (tile_rows, hidden), lambda l, b: (l * b_tiles + b, 0)
            ),
            pl.BlockSpec((out_vocab, hidden), lambda l, b: (0, 0)),
            pl.BlockSpec((out_vocab, 1), lambda l, b: (0, 0)),
        ],
        out_specs=pl.BlockSpec(
            (1, out_vocab, tile_rows), lambda l, b: (l, 0, b)
        ),
        out_shape=jax.ShapeDtypeStruct(
            (out_len, out_vocab, batch), jnp.float32
        ),
    )(gathered, jnp.transpose(Wout), bout.reshape(out_vocab, 1))

    return jnp.transpose(outT, (2, 0, 1))


# speedup vs baseline: 3.9508x; 1.8813x over previous
import functools

import jax
import jax.numpy as jnp
from jax import lax
from jax.experimental import pallas as pl
from jax.experimental.pallas import tpu as pltpu
from jax.experimental.pallas import tpu_sc as plsc

CHUNK = 128
NBUF = 4


def _make_sc_gather(n_chunks, chunks_c0, chunks_c1, hidden, nc, ns):
    mesh = plsc.VectorSubcoreMesh(core_axis_name="c", subcore_axis_name="s")
    assert ns * (chunks_c0 + chunks_c1) == n_chunks

    @functools.partial(
        pl.kernel,
        mesh=mesh,
        out_type=jax.ShapeDtypeStruct((n_chunks * CHUNK, hidden), jnp.float32),
        scratch_types=[
            pltpu.VMEM((chunks_c0 * CHUNK,), jnp.int32),
            pltpu.VMEM((NBUF, CHUNK, hidden), jnp.float32),
            pltpu.SemaphoreType.DMA,
        ],
    )
    def sc_gather(idx_hbm, table_hbm, out_hbm, idx_v, rows_v, sem):
        c = lax.axis_index("c")
        s = lax.axis_index("s")
        my_chunks = jnp.where(c == 0, chunks_c0, chunks_c1)
        chunk_base = jnp.where(
            c == 0, s * chunks_c0, ns * chunks_c0 + s * chunks_c1
        )
        row_base = pl.multiple_of(chunk_base * CHUNK, CHUNK)
        pltpu.sync_copy(
            idx_hbm.at[pl.ds(row_base, chunks_c0 * CHUNK)], idx_v
        )

        def copy_j(j):
            return pltpu.make_async_copy(
                table_hbm.at[idx_v.at[pl.ds(j * CHUNK, CHUNK)]],
                rows_v.at[j % NBUF],
                sem,
            )

        for step in range(chunks_c0 + NBUF - 1):
            if step < chunks_c0:

                @pl.when(step < my_chunks)
                def _(j=step):
                    copy_j(j).start()

            jj = step - (NBUF - 1)
            if 0 <= jj < chunks_c0:

                @pl.when(jj < my_chunks)
                def _(j=jj):
                    copy_j(j).wait()
                    out_row = pl.multiple_of(
                        (chunk_base + j) * CHUNK, CHUNK
                    )
                    pltpu.sync_copy(
                        rows_v.at[j % NBUF],
                        out_hbm.at[pl.ds(out_row, CHUNK)],
                    )

    return sc_gather


def _relu_ff_t_body(xt_ref, w2_ref, b2_ref, o_ref):
    h = lax.dot_general(
        xt_ref[...],
        w2_ref[...],
        (((0,), (0,)), ((), ())),
        preferred_element_type=jnp.float32,
    )
    o_ref[...] = jnp.maximum(h + b2_ref[...], 0.0)


def _out_softmax_t_body(h_ref, woutt_ref, boutt_ref, o_ref):
    l_blk, _, batch = o_ref.shape
    for i in range(l_blk):
        logits = lax.dot_general(
            woutt_ref[...],
            h_ref[pl.ds(i * batch, batch), :],
            (((1,), (1,)), ((), ())),
            preferred_element_type=jnp.float32,
        )
        logits = logits + boutt_ref[...]
        m = jnp.max(logits, axis=0, keepdims=True)
        e = jnp.exp(logits - m)
        o_ref[i] = e * (1.0 / jnp.sum(e, axis=0, keepdims=True))


def kernel(input_seq, output_seq, emb, W1, b1, W2, b2, Wout, bout):
    del input_seq, W1, b1

    batch, out_len = output_seq.shape
    n_rows = batch * out_len
    vocab, embed_dim = emb.shape
    hidden = W2.shape[1]
    out_vocab = Wout.shape[1]

    info = plsc.get_sparse_core_info()
    nc, ns = info.num_cores, info.num_subcores
    nw = nc * ns

    embT = jnp.transpose(emb)
    pre_rows = 4096
    htable = pl.pallas_call(
        _relu_ff_t_body,
        grid=(-(-vocab // pre_rows),),
        in_specs=[
            pl.BlockSpec((embed_dim, pre_rows), lambda i: (0, i)),
            pl.BlockSpec((embed_dim, hidden), lambda i: (0, 0)),
            pl.BlockSpec((1, hidden), lambda i: (0, 0)),
        ],
        out_specs=pl.BlockSpec((pre_rows, hidden), lambda i: (i, 0)),
        out_shape=jax.ShapeDtypeStruct((vocab, hidden), jnp.float32),
    )(embT, W2, b2.reshape(1, hidden))

    idx = jnp.transpose(output_seq).reshape(-1).astype(jnp.int32)
    n_chunks = n_rows // CHUNK
    per_pair = n_chunks // ns
    chunks_c0 = (2 * per_pair + 1) // 3
    chunks_c1 = per_pair - chunks_c0
    pad_chunks = max(
        0, ns * chunks_c0 + (ns - 1) * chunks_c1 + chunks_c0 - n_chunks
    )
    idx = jnp.pad(idx, (0, pad_chunks * CHUNK))

    gathered = _make_sc_gather(n_chunks, chunks_c0, chunks_c1, hidden, nc, ns)(
        idx, htable
    )

    l_blk = 2
    outT = pl.pallas_call(
        _out_softmax_t_body,
        grid=(out_len // l_blk,),
        in_specs=[
            pl.BlockSpec((l_blk * batch, hidden), lambda l: (l, 0)),
            pl.BlockSpec((out_vocab, hidden), lambda l: (0, 0)),
            pl.BlockSpec((out_vocab, 1), lambda l: (0, 0)),
        ],
        out_specs=pl.BlockSpec(
            (l_blk, out_vocab, batch), lambda l: (l, 0, 0)
        ),
        out_shape=jax.ShapeDtypeStruct(
            (out_len, out_vocab, batch), jnp.float32
        ),
        compiler_params=pltpu.CompilerParams(
            vmem_limit_bytes=100 * 1024 * 1024
        ),
    )(gathered, jnp.transpose(Wout), bout.reshape(out_vocab, 1))

    return jnp.transpose(outT, (2, 0, 1))


# speedup vs baseline: 4.1360x; 1.0469x over previous
import functools

import jax
import jax.numpy as jnp
from jax import lax
from jax.experimental import pallas as pl
from jax.experimental.pallas import tpu as pltpu
from jax.experimental.pallas import tpu_sc as plsc

CHUNK = 128
NBUF = 4


def _make_sc_gather(n_chunks, chunks_c0, chunks_c1, hidden, nc, ns):
    mesh = plsc.VectorSubcoreMesh(core_axis_name="c", subcore_axis_name="s")
    assert ns * (chunks_c0 + chunks_c1) == n_chunks

    @functools.partial(
        pl.kernel,
        mesh=mesh,
        out_type=jax.ShapeDtypeStruct((n_chunks * CHUNK, hidden), jnp.float32),
        scratch_types=[
            pltpu.VMEM((chunks_c0 * CHUNK,), jnp.int32),
            pltpu.VMEM((NBUF, CHUNK, hidden), jnp.float32),
            pltpu.SemaphoreType.DMA,
        ],
    )
    def sc_gather(idx_hbm, table_hbm, out_hbm, idx_v, rows_v, sem):
        c = lax.axis_index("c")
        s = lax.axis_index("s")
        my_chunks = jnp.where(c == 0, chunks_c0, chunks_c1)
        chunk_base = jnp.where(
            c == 0, s * chunks_c0, ns * chunks_c0 + s * chunks_c1
        )
        row_base = pl.multiple_of(chunk_base * CHUNK, CHUNK)
        pltpu.sync_copy(
            idx_hbm.at[pl.ds(row_base, chunks_c0 * CHUNK)], idx_v
        )

        def copy_j(j):
            return pltpu.make_async_copy(
                table_hbm.at[idx_v.at[pl.ds(j * CHUNK, CHUNK)]],
                rows_v.at[j % NBUF],
                sem,
            )

        for step in range(chunks_c0 + NBUF - 1):
            if step < chunks_c0:

                @pl.when(step < my_chunks)
                def _(j=step):
                    copy_j(j).start()

            jj = step - (NBUF - 1)
            if 0 <= jj < chunks_c0:

                @pl.when(jj < my_chunks)
                def _(j=jj):
                    copy_j(j).wait()
                    out_row = pl.multiple_of(
                        (chunk_base + j) * CHUNK, CHUNK
                    )
                    pltpu.sync_copy(
                        rows_v.at[j % NBUF],
                        out_hbm.at[pl.ds(out_row, CHUNK)],
                    )

    return sc_gather


def _relu_ff_t_body(xt_ref, w2_ref, b2_ref, o_ref):
    h = lax.dot_general(
        xt_ref[...],
        w2_ref[...],
        (((0,), (0,)), ((), ())),
        preferred_element_type=jnp.float32,
    )
    o_ref[...] = jnp.maximum(h + b2_ref[...], 0.0)


def _out_softmax_t_body(h_ref, woutt_ref, boutt_ref, o_ref):
    l_blk, _, batch = o_ref.shape
    for i in range(l_blk):
        logits = lax.dot_general(
            woutt_ref[...],
            h_ref[pl.ds(i * batch, batch), :],
            (((1,), (1,)), ((), ())),
            preferred_element_type=jnp.float32,
        )
        logits = logits + boutt_ref[...]
        m = jnp.max(logits, axis=0, keepdims=True)
        e = jnp.exp(logits - m)
        o_ref[i] = e * (1.0 / jnp.sum(e, axis=0, keepdims=True))


def kernel(input_seq, output_seq, emb, W1, b1, W2, b2, Wout, bout):
    del input_seq, W1, b1

    batch, out_len = output_seq.shape
    n_rows = batch * out_len
    vocab, embed_dim = emb.shape
    hidden = W2.shape[1]
    out_vocab = Wout.shape[1]

    info = plsc.get_sparse_core_info()
    nc, ns = info.num_cores, info.num_subcores
    nw = nc * ns

    embT = jnp.transpose(emb)
    pre_rows = 8192
    htable = pl.pallas_call(
        _relu_ff_t_body,
        grid=(-(-vocab // pre_rows),),
        in_specs=[
            pl.BlockSpec((embed_dim, pre_rows), lambda i: (0, i)),
            pl.BlockSpec((embed_dim, hidden), lambda i: (0, 0)),
            pl.BlockSpec((1, hidden), lambda i: (0, 0)),
        ],
        out_specs=pl.BlockSpec((pre_rows, hidden), lambda i: (i, 0)),
        out_shape=jax.ShapeDtypeStruct((vocab, hidden), jnp.float32),
    )(embT, W2, b2.reshape(1, hidden))

    idx = jnp.transpose(output_seq).reshape(-1).astype(jnp.int32)
    n_chunks = n_rows // CHUNK
    per_pair = n_chunks // ns
    chunks_c0 = (2 * per_pair + 1) // 3 - 1
    chunks_c1 = per_pair - chunks_c0
    pad_chunks = max(
        0, ns * chunks_c0 + (ns - 1) * chunks_c1 + chunks_c0 - n_chunks
    )
    idx = jnp.pad(idx, (0, pad_chunks * CHUNK))

    gathered = _make_sc_gather(n_chunks, chunks_c0, chunks_c1, hidden, nc, ns)(
        idx, htable
    )

    l_blk = 2
    outT = pl.pallas_call(
        _out_softmax_t_body,
        grid=(out_len // l_blk,),
        in_specs=[
            pl.BlockSpec((l_blk * batch, hidden), lambda l: (l, 0)),
            pl.BlockSpec((out_vocab, hidden), lambda l: (0, 0)),
            pl.BlockSpec((out_vocab, 1), lambda l: (0, 0)),
        ],
        out_specs=pl.BlockSpec(
            (l_blk, out_vocab, batch), lambda l: (l, 0, 0)
        ),
        out_shape=jax.ShapeDtypeStruct(
            (out_len, out_vocab, batch), jnp.float32
        ),
        compiler_params=pltpu.CompilerParams(
            vmem_limit_bytes=100 * 1024 * 1024
        ),
    )(gathered, jnp.transpose(Wout), bout.reshape(out_vocab, 1))

    return jnp.transpose(outT, (2, 0, 1))


# speedup vs baseline: 4.1562x; 1.0049x over previous
import functools

import jax
import jax.numpy as jnp
from jax import lax
from jax.experimental import pallas as pl
from jax.experimental.pallas import tpu as pltpu
from jax.experimental.pallas import tpu_sc as plsc

CHUNK = 128
NBUF = 4


def _make_sc_gather(n_chunks, chunks_c0, chunks_c1, hidden, nc, ns):
    mesh = plsc.VectorSubcoreMesh(core_axis_name="c", subcore_axis_name="s")
    assert ns * (chunks_c0 + chunks_c1) == n_chunks

    @functools.partial(
        pl.kernel,
        mesh=mesh,
        out_type=jax.ShapeDtypeStruct((n_chunks * CHUNK, hidden), jnp.float32),
        scratch_types=[
            pltpu.VMEM((chunks_c0 * CHUNK,), jnp.int32),
            pltpu.VMEM((NBUF, CHUNK, hidden), jnp.float32),
            pltpu.SemaphoreType.DMA,
        ],
    )
    def sc_gather(idx_hbm, table_hbm, out_hbm, idx_v, rows_v, sem):
        c = lax.axis_index("c")
        s = lax.axis_index("s")
        my_chunks = jnp.where(c == 0, chunks_c0, chunks_c1)
        chunk_base = jnp.where(
            c == 0, s * chunks_c0, ns * chunks_c0 + s * chunks_c1
        )
        row_base = pl.multiple_of(chunk_base * CHUNK, CHUNK)
        pltpu.sync_copy(
            idx_hbm.at[pl.ds(row_base, chunks_c0 * CHUNK)], idx_v
        )

        def copy_j(j):
            return pltpu.make_async_copy(
                table_hbm.at[idx_v.at[pl.ds(j * CHUNK, CHUNK)]],
                rows_v.at[j % NBUF],
                sem,
            )

        for step in range(chunks_c0 + NBUF - 1):
            if step < chunks_c0:

                @pl.when(step < my_chunks)
                def _(j=step):
                    copy_j(j).start()

            jj = step - (NBUF - 1)
            if 0 <= jj < chunks_c0:

                @pl.when(jj < my_chunks)
                def _(j=jj):
                    copy_j(j).wait()
                    out_row = pl.multiple_of(
                        (chunk_base + j) * CHUNK, CHUNK
                    )
                    pltpu.sync_copy(
                        rows_v.at[j % NBUF],
                        out_hbm.at[pl.ds(out_row, CHUNK)],
                    )

    return sc_gather


def _relu_ff_t_body(xt_ref, w2_ref, b2_ref, o_ref):
    h = lax.dot_general(
        xt_ref[...],
        w2_ref[...],
        (((0,), (0,)), ((), ())),
        preferred_element_type=jnp.float32,
    )
    o_ref[...] = jnp.maximum(h + b2_ref[...], 0.0)


def _out_softmax_t_body(h_ref, woutt_ref, boutt_ref, o_ref):
    l_blk, _, batch = o_ref.shape
    for i in range(l_blk):
        logits = lax.dot_general(
            woutt_ref[...],
            h_ref[pl.ds(i * batch, batch), :],
            (((1,), (1,)), ((), ())),
            preferred_element_type=jnp.float32,
        )
        logits = logits + boutt_ref[...]
        m = jnp.max(logits, axis=0, keepdims=True)
        e = jnp.exp(logits - m)
        o_ref[i] = e * (1.0 / jnp.sum(e, axis=0, keepdims=True))


def kernel(input_seq, output_seq, emb, W1, b1, W2, b2, Wout, bout):
    del input_seq, W1, b1

    batch, out_len = output_seq.shape
    n_rows = batch * out_len
    vocab, embed_dim = emb.shape
    hidden = W2.shape[1]
    out_vocab = Wout.shape[1]

    info = plsc.get_sparse_core_info()
    nc, ns = info.num_cores, info.num_subcores
    nw = nc * ns

    embT = jnp.transpose(emb)
    pre_rows = 12544
    htable = pl.pallas_call(
        _relu_ff_t_body,
        grid=(-(-vocab // pre_rows),),
        in_specs=[
            pl.BlockSpec((embed_dim, pre_rows), lambda i: (0, i)),
            pl.BlockSpec((embed_dim, hidden), lambda i: (0, 0)),
            pl.BlockSpec((1, hidden), lambda i: (0, 0)),
        ],
        out_specs=pl.BlockSpec((pre_rows, hidden), lambda i: (i, 0)),
        out_shape=jax.ShapeDtypeStruct((vocab, hidden), jnp.float32),
    )(embT, W2, b2.reshape(1, hidden))

    idx = jnp.transpose(output_seq).reshape(-1).astype(jnp.int32)
    n_chunks = n_rows // CHUNK
    per_pair = n_chunks // ns
    chunks_c0 = (2 * per_pair + 1) // 3 - 1
    chunks_c1 = per_pair - chunks_c0
    pad_chunks = max(
        0, ns * chunks_c0 + (ns - 1) * chunks_c1 + chunks_c0 - n_chunks
    )
    idx = jnp.pad(idx, (0, pad_chunks * CHUNK))

    gathered = _make_sc_gather(n_chunks, chunks_c0, chunks_c1, hidden, nc, ns)(
        idx, htable
    )

    l_blk = 2
    outT = pl.pallas_call(
        _out_softmax_t_body,
        grid=(out_len // l_blk,),
        in_specs=[
            pl.BlockSpec((l_blk * batch, hidden), lambda l: (l, 0)),
            pl.BlockSpec((out_vocab, hidden), lambda l: (0, 0)),
            pl.BlockSpec((out_vocab, 1), lambda l: (0, 0)),
        ],
        out_specs=pl.BlockSpec(
            (l_blk, out_vocab, batch), lambda l: (l, 0, 0)
        ),
        out_shape=jax.ShapeDtypeStruct(
            (out_len, out_vocab, batch), jnp.float32
        ),
        compiler_params=pltpu.CompilerParams(
            vmem_limit_bytes=100 * 1024 * 1024
        ),
    )(gathered, jnp.transpose(Wout), bout.reshape(out_vocab, 1))

    return jnp.transpose(outT, (2, 0, 1))
